# Initial kernel scaffold; baseline (speedup 1.0000x reference)
import functools

import jax
import jax.numpy as jnp
from jax import lax
from jax.experimental import pallas as pl
from jax.experimental.pallas import tpu as pltpu
from jax.experimental.pallas import tpu_sc as plsc

N = 10000
E = 320000
D_IN = 128
D_HID = 16
D_OUT = 40
D_OUTP = 48

NC = 2
NS = 16
NW = NC * NS
EPW = E // NW
K = 100
NCH = EPW // K
RPT = N // NS

_MESH = plsc.VectorSubcoreMesh(core_axis_name="c", subcore_axis_name="s")


def _make_sc_deg():
    @functools.partial(
        pl.kernel,
        out_type=jax.ShapeDtypeStruct((NC, N, D_HID), jnp.float32),
        mesh=_MESH,
        scratch_types=[
            pltpu.VMEM((NCH, K), jnp.int32),
            pltpu.VMEM((K, D_HID), jnp.float32),
            pltpu.VMEM_SHARED((N, D_HID), jnp.float32),
        ],
    )
    def deg_kernel(dst_hbm, zero_hbm, out_hbm, dst_v, ones_v, acc):
        c = lax.axis_index("c")
        s = lax.axis_index("s")
        wid = c * NS + s
        pltpu.sync_copy(dst_hbm.at[wid], dst_v)

        @pl.loop(0, K)
        def _(i):
            ones_v[i] = jnp.ones((D_HID,), jnp.float32)

        pltpu.sync_copy(zero_hbm.at[pl.ds(s * RPT, RPT)],
                        acc.at[pl.ds(s * RPT, RPT)])
        plsc.subcore_barrier()

        @pl.loop(0, NCH)
        def _(j):
            pltpu.sync_copy(ones_v, acc.at[dst_v.at[j]], add=True)

        plsc.subcore_barrier()
        pltpu.sync_copy(acc.at[pl.ds(s * RPT, RPT)],
                        out_hbm.at[c, pl.ds(s * RPT, RPT)])

    return deg_kernel


def _make_sc_agg(d):
    @functools.partial(
        pl.kernel,
        out_type=jax.ShapeDtypeStruct((NC, N, d), jnp.float32),
        mesh=_MESH,
        scratch_types=[
            pltpu.VMEM((NCH, K), jnp.int32),
            pltpu.VMEM((NCH, K), jnp.int32),
            pltpu.VMEM((K, d), jnp.float32),
            pltpu.VMEM((K, d), jnp.float32),
            pltpu.VMEM_SHARED((N, d), jnp.float32),
            pltpu.SemaphoreType.DMA,
            pltpu.SemaphoreType.DMA,
        ],
    )
    def agg_kernel(val_hbm, src_hbm, dst_hbm, zero_hbm, out_hbm,
                   src_v, dst_v, buf_a, buf_b, acc, sem_a, sem_b):
        c = lax.axis_index("c")
        s = lax.axis_index("s")
        wid = c * NS + s
        pltpu.sync_copy(src_hbm.at[wid], src_v)
        pltpu.sync_copy(dst_hbm.at[wid], dst_v)
        pltpu.sync_copy(zero_hbm.at[pl.ds(s * RPT, RPT)],
                        acc.at[pl.ds(s * RPT, RPT)])
        plsc.subcore_barrier()

        pltpu.async_copy(val_hbm.at[src_v.at[0]], buf_a, sem_a)

        @pl.loop(0, NCH, step=2)
        def _(j):
            pltpu.async_copy(val_hbm.at[src_v.at[j + 1]], buf_b, sem_b)
            pltpu.make_async_copy(val_hbm.at[src_v.at[j]], buf_a, sem_a).wait()
            pltpu.sync_copy(buf_a, acc.at[dst_v.at[j]], add=True)

            @pl.when(j + 2 < NCH)
            def _():
                pltpu.async_copy(val_hbm.at[src_v.at[j + 2]], buf_a, sem_a)

            pltpu.make_async_copy(val_hbm.at[src_v.at[j + 1]], buf_b,
                                  sem_b).wait()
            pltpu.sync_copy(buf_b, acc.at[dst_v.at[j + 1]], add=True)

        plsc.subcore_barrier()
        pltpu.sync_copy(acc.at[pl.ds(s * RPT, RPT)],
                        out_hbm.at[c, pl.ds(s * RPT, RPT)])

    return agg_kernel


_SC_DEG = _make_sc_deg()
_SC_AGG_H = _make_sc_agg(D_HID)
_SC_AGG_O = _make_sc_agg(D_OUTP)


_BR = 2000
_G = N // _BR


def _tc_a_body(x_ref, w1_ref, dega_ref, xwp_ref, dinv_ref):
    xw = jnp.dot(x_ref[...], w1_ref[...], preferred_element_type=jnp.float32)
    deg = 1.0 + dega_ref[0] + dega_ref[1]
    dinv = lax.rsqrt(deg)
    xwp_ref[...] = xw * dinv
    dinv_ref[...] = dinv


def _tc_a(x, w1, dega):
    return pl.pallas_call(
        _tc_a_body,
        grid=(_G,),
        in_specs=[
            pl.BlockSpec((_BR, D_IN), lambda i: (i, 0)),
            pl.BlockSpec((D_IN, D_HID), lambda i: (0, 0)),
            pl.BlockSpec((NC, _BR, D_HID), lambda i: (0, i, 0)),
        ],
        out_specs=[
            pl.BlockSpec((_BR, D_HID), lambda i: (i, 0)),
            pl.BlockSpec((_BR, D_HID), lambda i: (i, 0)),
        ],
        out_shape=[
            jax.ShapeDtypeStruct((N, D_HID), jnp.float32),
            jax.ShapeDtypeStruct((N, D_HID), jnp.float32),
        ],
    )(x, w1, dega)


def _tc_b_body(agg_ref, xwp_ref, dinv_ref, b1_ref, w2_ref, hwp_ref):
    dinv = dinv_ref[...]
    pre = (agg_ref[0] + agg_ref[1] + xwp_ref[...]) * dinv + b1_ref[...]
    h = jnp.where(pre > 0, pre, jnp.expm1(pre))
    hw = jnp.dot(h, w2_ref[...], preferred_element_type=jnp.float32)
    dinv_o = jnp.broadcast_to(dinv[:, 0:1], (_BR, D_OUTP))
    hwp_ref[...] = hw * dinv_o


def _tc_b(agg1, xwp, dinv, b1, w2p):
    return pl.pallas_call(
        _tc_b_body,
        grid=(_G,),
        in_specs=[
            pl.BlockSpec((NC, _BR, D_HID), lambda i: (0, i, 0)),
            pl.BlockSpec((_BR, D_HID), lambda i: (i, 0)),
            pl.BlockSpec((_BR, D_HID), lambda i: (i, 0)),
            pl.BlockSpec((1, D_HID), lambda i: (0, 0)),
            pl.BlockSpec((D_HID, D_OUTP), lambda i: (0, 0)),
        ],
        out_specs=pl.BlockSpec((_BR, D_OUTP), lambda i: (i, 0)),
        out_shape=jax.ShapeDtypeStruct((N, D_OUTP), jnp.float32),
    )(agg1, xwp, dinv, b1, w2p)


def _tc_c_body(agg_ref, hwp_ref, dinv_ref, b2_ref, o_ref):
    dinv_o = jnp.broadcast_to(dinv_ref[..., 0:1], (_BR, D_OUTP))
    o = (agg_ref[0] + agg_ref[1] + hwp_ref[...]) * dinv_o + b2_ref[...]
    col = lax.broadcasted_iota(jnp.int32, (_BR, D_OUTP), 1)
    valid = col < D_OUT
    om = jnp.where(valid, o, jnp.float32(-1e30))
    m = jnp.max(om, axis=1, keepdims=True)
    ex = jnp.where(valid, jnp.exp(o - m), 0.0)
    lse = jnp.log(jnp.sum(ex, axis=1, keepdims=True))
    o_ref[...] = o - m - lse


def _tc_c(agg2, hwp, dinv, b2p):
    return pl.pallas_call(
        _tc_c_body,
        grid=(_G,),
        in_specs=[
            pl.BlockSpec((NC, _BR, D_OUTP), lambda i: (0, i, 0)),
            pl.BlockSpec((_BR, D_OUTP), lambda i: (i, 0)),
            pl.BlockSpec((_BR, D_HID), lambda i: (i, 0)),
            pl.BlockSpec((1, D_OUTP), lambda i: (0, 0)),
        ],
        out_specs=pl.BlockSpec((_BR, D_OUTP), lambda i: (i, 0)),
        out_shape=jax.ShapeDtypeStruct((N, D_OUTP), jnp.float32),
    )(agg2, hwp, dinv, b2p)


@jax.jit
def kernel(node_feature, edge_index, W1, b1, W2, b2):
    src3 = edge_index[0].reshape(NW, NCH, K)
    dst3 = edge_index[1].reshape(NW, NCH, K)
    z_h = jnp.zeros((N, D_HID), jnp.float32)
    z_o = jnp.zeros((N, D_OUTP), jnp.float32)
    w2p = jnp.pad(W2, ((0, 0), (0, D_OUTP - D_OUT)))
    b1r = b1.reshape(1, D_HID)
    b2p = jnp.pad(b2, (0, D_OUTP - D_OUT)).reshape(1, D_OUTP)

    dega = _SC_DEG(dst3, z_h)
    xwp, dinv = _tc_a(node_feature, W1, dega)
    agg1 = _SC_AGG_H(xwp, src3, dst3, z_h)
    hwp = _tc_b(agg1, xwp, dinv, b1r, w2p)
    agg2 = _SC_AGG_O(hwp, src3, dst3, z_o)
    o = _tc_c(agg2, hwp, dinv, b2p)
    return o[:, :D_OUT]


# speedup vs baseline: 40.6551x; 40.6551x over previous
import functools

import jax
import jax.numpy as jnp
from jax import lax
from jax.experimental import pallas as pl
from jax.experimental.pallas import tpu as pltpu
from jax.experimental.pallas import tpu_sc as plsc

N = 10000
E = 320000
D_IN = 128
D_HID = 16
D_OUT = 40
D_OUTP = 48

NC = 2
NS = 16
NW = NC * NS
EPW = E // NW
K = 100
NCH = EPW // K
RB = 624
TAIL0 = NS * RB
TAILN = N - TAIL0

_MESH = plsc.VectorSubcoreMesh(core_axis_name="c", subcore_axis_name="s")
_SC_PARAMS = pltpu.CompilerParams(use_tc_tiling_on_sc=False)


def _tile_rows_copy(src, dst, s):
    pltpu.sync_copy(src.at[pl.ds(s * RB, RB)], dst.at[pl.ds(s * RB, RB)])

    @pl.when(s == 0)
    def _():
        pltpu.sync_copy(src.at[pl.ds(TAIL0, TAILN)], dst.at[pl.ds(TAIL0, TAILN)])


def _make_sc_deg():
    @functools.partial(
        pl.kernel,
        out_type=jax.ShapeDtypeStruct((NC, N, D_HID), jnp.float32),
        mesh=_MESH,
        scratch_types=[
            pltpu.VMEM((NCH, K), jnp.int32),
            pltpu.VMEM((K, D_HID), jnp.float32),
            pltpu.VMEM_SHARED((N, D_HID), jnp.float32),
        ],
        compiler_params=_SC_PARAMS,
    )
    def deg_kernel(dst_hbm, zero_hbm, out_hbm, dst_v, ones_v, acc):
        c = lax.axis_index("c")
        s = lax.axis_index("s")
        wid = c * NS + s
        pltpu.sync_copy(dst_hbm.at[wid], dst_v)

        @pl.loop(0, K)
        def _(i):
            ones_v[i] = jnp.ones((D_HID,), jnp.float32)

        _tile_rows_copy(zero_hbm, acc, s)
        plsc.subcore_barrier()

        @pl.loop(0, NCH)
        def _(j):
            pltpu.sync_copy(ones_v, acc.at[dst_v.at[j]], add=True)

        plsc.subcore_barrier()
        _tile_rows_copy(acc, out_hbm.at[c], s)

    return deg_kernel


def _make_sc_agg(d):
    @functools.partial(
        pl.kernel,
        out_type=jax.ShapeDtypeStruct((NC, N, d), jnp.float32),
        mesh=_MESH,
        scratch_types=[
            pltpu.VMEM((NCH, K), jnp.int32),
            pltpu.VMEM((NCH, K), jnp.int32),
            pltpu.VMEM((K, d), jnp.float32),
            pltpu.VMEM((K, d), jnp.float32),
            pltpu.VMEM_SHARED((N, d), jnp.float32),
            pltpu.SemaphoreType.DMA,
            pltpu.SemaphoreType.DMA,
        ],
        compiler_params=_SC_PARAMS,
    )
    def agg_kernel(val_hbm, src_hbm, dst_hbm, zero_hbm, out_hbm,
                   src_v, dst_v, buf_a, buf_b, acc, sem_a, sem_b):
        c = lax.axis_index("c")
        s = lax.axis_index("s")
        wid = c * NS + s
        pltpu.sync_copy(src_hbm.at[wid], src_v)
        pltpu.sync_copy(dst_hbm.at[wid], dst_v)
        _tile_rows_copy(zero_hbm, acc, s)
        plsc.subcore_barrier()

        pltpu.async_copy(val_hbm.at[src_v.at[0]], buf_a, sem_a)

        @pl.loop(0, NCH, step=2)
        def _(j):
            pltpu.async_copy(val_hbm.at[src_v.at[j + 1]], buf_b, sem_b)
            pltpu.make_async_copy(val_hbm.at[src_v.at[j]], buf_a, sem_a).wait()
            pltpu.sync_copy(buf_a, acc.at[dst_v.at[j]], add=True)

            @pl.when(j + 2 < NCH)
            def _():
                pltpu.async_copy(val_hbm.at[src_v.at[j + 2]], buf_a, sem_a)

            pltpu.make_async_copy(val_hbm.at[src_v.at[j + 1]], buf_b,
                                  sem_b).wait()
            pltpu.sync_copy(buf_b, acc.at[dst_v.at[j + 1]], add=True)

        plsc.subcore_barrier()
        _tile_rows_copy(acc, out_hbm.at[c], s)

    return agg_kernel


_SC_DEG = _make_sc_deg()
_SC_AGG_H = _make_sc_agg(D_HID)
_SC_AGG_O = _make_sc_agg(D_OUTP)


_BR = 2000
_G = N // _BR


def _tc_a_body(x_ref, w1_ref, dega_ref, xwp_ref, dinv_ref):
    xw = jnp.dot(x_ref[...], w1_ref[...], preferred_element_type=jnp.float32)
    deg = 1.0 + dega_ref[0] + dega_ref[1]
    dinv = lax.rsqrt(deg)
    xwp_ref[...] = xw * dinv
    dinv_ref[...] = dinv


def _tc_a(x, w1, dega):
    return pl.pallas_call(
        _tc_a_body,
        grid=(_G,),
        in_specs=[
            pl.BlockSpec((_BR, D_IN), lambda i: (i, 0)),
            pl.BlockSpec((D_IN, D_HID), lambda i: (0, 0)),
            pl.BlockSpec((NC, _BR, D_HID), lambda i: (0, i, 0)),
        ],
        out_specs=[
            pl.BlockSpec((_BR, D_HID), lambda i: (i, 0)),
            pl.BlockSpec((_BR, D_HID), lambda i: (i, 0)),
        ],
        out_shape=[
            jax.ShapeDtypeStruct((N, D_HID), jnp.float32),
            jax.ShapeDtypeStruct((N, D_HID), jnp.float32),
        ],
    )(x, w1, dega)


def _tc_b_body(agg_ref, xwp_ref, dinv_ref, b1_ref, w2_ref, hwp_ref):
    dinv = dinv_ref[...]
    pre = (agg_ref[0] + agg_ref[1] + xwp_ref[...]) * dinv + b1_ref[...]
    h = jnp.where(pre > 0, pre, jnp.exp(jnp.minimum(pre, 0.0)) - 1.0)
    hw = jnp.dot(h, w2_ref[...], preferred_element_type=jnp.float32)
    dinv_o = jnp.broadcast_to(dinv[:, 0:1], (_BR, D_OUTP))
    hwp_ref[...] = hw * dinv_o


def _tc_b(agg1, xwp, dinv, b1, w2p):
    return pl.pallas_call(
        _tc_b_body,
        grid=(_G,),
        in_specs=[
            pl.BlockSpec((NC, _BR, D_HID), lambda i: (0, i, 0)),
            pl.BlockSpec((_BR, D_HID), lambda i: (i, 0)),
            pl.BlockSpec((_BR, D_HID), lambda i: (i, 0)),
            pl.BlockSpec((1, D_HID), lambda i: (0, 0)),
            pl.BlockSpec((D_HID, D_OUTP), lambda i: (0, 0)),
        ],
        out_specs=pl.BlockSpec((_BR, D_OUTP), lambda i: (i, 0)),
        out_shape=jax.ShapeDtypeStruct((N, D_OUTP), jnp.float32),
    )(agg1, xwp, dinv, b1, w2p)


def _tc_c_body(agg_ref, hwp_ref, dinv_ref, b2_ref, o_ref):
    dinv_o = jnp.broadcast_to(dinv_ref[..., 0:1], (_BR, D_OUTP))
    o = (agg_ref[0] + agg_ref[1] + hwp_ref[...]) * dinv_o + b2_ref[...]
    col = lax.broadcasted_iota(jnp.int32, (_BR, D_OUTP), 1)
    valid = col < D_OUT
    om = jnp.where(valid, o, jnp.float32(-1e30))
    m = jnp.max(om, axis=1, keepdims=True)
    ex = jnp.where(valid, jnp.exp(o - m), 0.0)
    lse = jnp.log(jnp.sum(ex, axis=1, keepdims=True))
    o_ref[...] = o - m - lse


def _tc_c(agg2, hwp, dinv, b2p):
    return pl.pallas_call(
        _tc_c_body,
        grid=(_G,),
        in_specs=[
            pl.BlockSpec((NC, _BR, D_OUTP), lambda i: (0, i, 0)),
            pl.BlockSpec((_BR, D_OUTP), lambda i: (i, 0)),
            pl.BlockSpec((_BR, D_HID), lambda i: (i, 0)),
            pl.BlockSpec((1, D_OUTP), lambda i: (0, 0)),
        ],
        out_specs=pl.BlockSpec((_BR, D_OUTP), lambda i: (i, 0)),
        out_shape=jax.ShapeDtypeStruct((N, D_OUTP), jnp.float32),
    )(agg2, hwp, dinv, b2p)


@jax.jit
def kernel(node_feature, edge_index, W1, b1, W2, b2):
    src3 = edge_index[0].reshape(NW, NCH, K)
    dst3 = edge_index[1].reshape(NW, NCH, K)
    z_h = jnp.zeros((N, D_HID), jnp.float32)
    z_o = jnp.zeros((N, D_OUTP), jnp.float32)
    w2p = jnp.pad(W2, ((0, 0), (0, D_OUTP - D_OUT)))
    b1r = b1.reshape(1, D_HID)
    b2p = jnp.pad(b2, (0, D_OUTP - D_OUT)).reshape(1, D_OUTP)

    dega = _SC_DEG(dst3, z_h)
    xwp, dinv = _tc_a(node_feature, W1, dega)
    agg1 = _SC_AGG_H(xwp, src3, dst3, z_h)
    hwp = _tc_b(agg1, xwp, dinv, b1r, w2p)
    agg2 = _SC_AGG_O(hwp, src3, dst3, z_o)
    o = _tc_c(agg2, hwp, dinv, b2p)
    return o[:, :D_OUT]


# speedup vs baseline: 48.2409x; 1.1866x over previous
import functools

import jax
import jax.numpy as jnp
from jax import lax
from jax.experimental import pallas as pl
from jax.experimental.pallas import tpu as pltpu
from jax.experimental.pallas import tpu_sc as plsc

N = 10000
E = 320000
D_IN = 128
D_HID = 16
D_OUT = 40
D_OUTP = 48

NC = 2
NS = 16
NW = NC * NS
EPW = E // NW
K = 100
NCH = EPW // K
RB = 624
TAIL0 = NS * RB
TAILN = N - TAIL0

_MESH = plsc.VectorSubcoreMesh(core_axis_name="c", subcore_axis_name="s")
_SC_PARAMS = pltpu.CompilerParams(use_tc_tiling_on_sc=False,
                                  needs_layout_passes=False)


def _tile_rows_copy(src, dst, s):
    pltpu.sync_copy(src.at[pl.ds(s * RB, RB)], dst.at[pl.ds(s * RB, RB)])

    @pl.when(s == 0)
    def _():
        pltpu.sync_copy(src.at[pl.ds(TAIL0, TAILN)], dst.at[pl.ds(TAIL0, TAILN)])


NR = N // 16


def _make_sc_deg():
    @functools.partial(
        pl.kernel,
        out_type=jax.ShapeDtypeStruct((NC, NR, 16), jnp.float32),
        mesh=_MESH,
        scratch_types=[
            pltpu.VMEM((EPW,), jnp.int32),
            pltpu.VMEM((NR, 16), jnp.float32),
            pltpu.VMEM((5, 125), jnp.int32),
            pltpu.VMEM_SHARED((NR, 16), jnp.float32),
        ],
        compiler_params=_SC_PARAMS,
    )
    def deg_kernel(dstf_hbm, iota_hbm, zero_hbm, out_hbm,
                   dstf_v, hist, iota_v, acc):
        c = lax.axis_index("c")
        s = lax.axis_index("s")
        wid = c * NS + s
        pltpu.sync_copy(dstf_hbm.at[wid], dstf_v)
        pltpu.sync_copy(iota_hbm, iota_v)

        @pl.loop(0, NR)
        def _(i):
            hist[i] = jnp.zeros((16,), jnp.float32)

        @pl.when(s == 0)
        def _():
            pltpu.sync_copy(zero_hbm, acc)
        plsc.subcore_barrier()

        ones16 = jnp.ones((16,), jnp.float32)

        @pl.loop(0, EPW, step=16)
        def _(i):
            idx = dstf_v[pl.ds(i, 16)]
            plsc.addupdate_scatter(
                hist, [lax.shift_right_logical(idx, 4), idx & 15], ones16)

        @pl.loop(0, 5)
        def _(r):
            pltpu.sync_copy(hist.at[pl.ds(r * 125, 125)],
                            acc.at[iota_v.at[r]], add=True)

        plsc.subcore_barrier()

        @pl.when(s == 0)
        def _():
            pltpu.sync_copy(acc, out_hbm.at[c])

    return deg_kernel


def _make_sc_agg(d):
    @functools.partial(
        pl.kernel,
        out_type=jax.ShapeDtypeStruct((NC, N, d), jnp.float32),
        mesh=_MESH,
        scratch_types=[
            pltpu.VMEM((NCH, K), jnp.int32),
            pltpu.VMEM((NCH, K), jnp.int32),
            [pltpu.VMEM((K, d), jnp.float32)] * 4,
            pltpu.VMEM_SHARED((N, d), jnp.float32),
            [pltpu.SemaphoreType.DMA] * 4,
            [pltpu.SemaphoreType.DMA] * 4,
        ],
        compiler_params=_SC_PARAMS,
    )
    def agg_kernel(val_hbm, src_hbm, dst_hbm, zero_hbm, out_hbm,
                   src_v, dst_v, bufs, acc, gsems, ssems):
        c = lax.axis_index("c")
        s = lax.axis_index("s")
        wid = c * NS + s
        pltpu.sync_copy(src_hbm.at[wid], src_v)
        pltpu.sync_copy(dst_hbm.at[wid], dst_v)
        _tile_rows_copy(zero_hbm, acc, s)
        plsc.subcore_barrier()

        for u in range(4):
            pltpu.async_copy(val_hbm.at[src_v.at[u]], bufs[u], gsems[u])

        @pl.loop(0, NCH, step=4)
        def _(j):
            for u in range(4):
                pltpu.make_async_copy(val_hbm.at[src_v.at[j + u]], bufs[u],
                                      gsems[u]).wait()
                pltpu.async_copy(bufs[u], acc.at[dst_v.at[j + u]], ssems[u],
                                 add=True)
            for u in range(4):
                @pl.when(j + 4 + u < NCH)
                def _(u=u):
                    pltpu.make_async_copy(bufs[u], acc.at[dst_v.at[j + u]],
                                          ssems[u]).wait()
                    pltpu.async_copy(val_hbm.at[src_v.at[j + 4 + u]], bufs[u],
                                     gsems[u])

        for u in range(4):
            pltpu.make_async_copy(bufs[u], acc.at[dst_v.at[0]],
                                  ssems[u]).wait()

        plsc.subcore_barrier()
        _tile_rows_copy(acc, out_hbm.at[c], s)

    return agg_kernel


_SC_DEG = _make_sc_deg()
_SC_AGG_H = _make_sc_agg(D_HID)
_SC_AGG_O = _make_sc_agg(D_OUTP)


_BR = 2000
_G = N // _BR


def _tc_a_body(x_ref, w1_ref, dega_ref, xwp_ref, dinv_ref):
    xw = jnp.dot(x_ref[...], w1_ref[...], preferred_element_type=jnp.float32)
    deg = 1.0 + dega_ref[0] + dega_ref[1]
    dinv = jnp.broadcast_to(lax.rsqrt(deg), (_BR, D_HID))
    xwp_ref[...] = xw * dinv
    dinv_ref[...] = dinv


def _tc_a(x, w1, dega):
    return pl.pallas_call(
        _tc_a_body,
        grid=(_G,),
        in_specs=[
            pl.BlockSpec((_BR, D_IN), lambda i: (i, 0)),
            pl.BlockSpec((D_IN, D_HID), lambda i: (0, 0)),
            pl.BlockSpec((NC, _BR, 1), lambda i: (0, i, 0)),
        ],
        out_specs=[
            pl.BlockSpec((_BR, D_HID), lambda i: (i, 0)),
            pl.BlockSpec((_BR, D_HID), lambda i: (i, 0)),
        ],
        out_shape=[
            jax.ShapeDtypeStruct((N, D_HID), jnp.float32),
            jax.ShapeDtypeStruct((N, D_HID), jnp.float32),
        ],
    )(x, w1, dega)


def _tc_b_body(agg_ref, xwp_ref, dinv_ref, b1_ref, w2_ref, hwp_ref):
    dinv = dinv_ref[...]
    pre = (agg_ref[0] + agg_ref[1] + xwp_ref[...]) * dinv + b1_ref[...]
    h = jnp.where(pre > 0, pre, jnp.exp(jnp.minimum(pre, 0.0)) - 1.0)
    hw = jnp.dot(h, w2_ref[...], preferred_element_type=jnp.float32)
    dinv_o = jnp.broadcast_to(dinv[:, 0:1], (_BR, D_OUTP))
    hwp_ref[...] = hw * dinv_o


def _tc_b(agg1, xwp, dinv, b1, w2p):
    return pl.pallas_call(
        _tc_b_body,
        grid=(_G,),
        in_specs=[
            pl.BlockSpec((NC, _BR, D_HID), lambda i: (0, i, 0)),
            pl.BlockSpec((_BR, D_HID), lambda i: (i, 0)),
            pl.BlockSpec((_BR, D_HID), lambda i: (i, 0)),
            pl.BlockSpec((1, D_HID), lambda i: (0, 0)),
            pl.BlockSpec((D_HID, D_OUTP), lambda i: (0, 0)),
        ],
        out_specs=pl.BlockSpec((_BR, D_OUTP), lambda i: (i, 0)),
        out_shape=jax.ShapeDtypeStruct((N, D_OUTP), jnp.float32),
    )(agg1, xwp, dinv, b1, w2p)


def _tc_c_body(agg_ref, hwp_ref, dinv_ref, b2_ref, o_ref):
    dinv_o = jnp.broadcast_to(dinv_ref[..., 0:1], (_BR, D_OUTP))
    o = (agg_ref[0] + agg_ref[1] + hwp_ref[...]) * dinv_o + b2_ref[...]
    col = lax.broadcasted_iota(jnp.int32, (_BR, D_OUTP), 1)
    valid = col < D_OUT
    om = jnp.where(valid, o, jnp.float32(-1e30))
    m = jnp.max(om, axis=1, keepdims=True)
    ex = jnp.where(valid, jnp.exp(o - m), 0.0)
    lse = jnp.log(jnp.sum(ex, axis=1, keepdims=True))
    o_ref[...] = o - m - lse


def _tc_c(agg2, hwp, dinv, b2p):
    return pl.pallas_call(
        _tc_c_body,
        grid=(_G,),
        in_specs=[
            pl.BlockSpec((NC, _BR, D_OUTP), lambda i: (0, i, 0)),
            pl.BlockSpec((_BR, D_OUTP), lambda i: (i, 0)),
            pl.BlockSpec((_BR, D_HID), lambda i: (i, 0)),
            pl.BlockSpec((1, D_OUTP), lambda i: (0, 0)),
        ],
        out_specs=pl.BlockSpec((_BR, D_OUTP), lambda i: (i, 0)),
        out_shape=jax.ShapeDtypeStruct((N, D_OUTP), jnp.float32),
    )(agg2, hwp, dinv, b2p)


@jax.jit
def kernel(node_feature, edge_index, W1, b1, W2, b2):
    src3 = edge_index[0].reshape(NW, NCH, K)
    dst3 = edge_index[1].reshape(NW, NCH, K)
    dstf = edge_index[1].reshape(NW, EPW)
    iota = jnp.arange(NR, dtype=jnp.int32).reshape(5, 125)
    z_h = jnp.zeros((N, D_HID), jnp.float32)
    z_o = jnp.zeros((N, D_OUTP), jnp.float32)
    z_r = jnp.zeros((NR, 16), jnp.float32)
    w2p = jnp.pad(W2, ((0, 0), (0, D_OUTP - D_OUT)))
    b1r = b1.reshape(1, D_HID)
    b2p = jnp.pad(b2, (0, D_OUTP - D_OUT)).reshape(1, D_OUTP)

    dega = _SC_DEG(dstf, iota, z_r).reshape(NC, N, 1)
    xwp, dinv = _tc_a(node_feature, W1, dega)
    agg1 = _SC_AGG_H(xwp, src3, dst3, z_h)
    hwp = _tc_b(agg1, xwp, dinv, b1r, w2p)
    agg2 = _SC_AGG_O(hwp, src3, dst3, z_o)
    o = _tc_c(agg2, hwp, dinv, b2p)
    return o[:, :D_OUT]
